# Initial kernel scaffold; baseline (speedup 1.0000x reference)
import jax
import jax.numpy as jnp
from jax.experimental import pallas as pl


def kernel(weights, indices, table):
    raise NotImplementedError("write your pallas kernel here")


# speedup vs baseline: 12.6397x; 12.6397x over previous
import functools

import jax
import jax.numpy as jnp
from jax import lax
from jax.experimental import pallas as pl
from jax.experimental.pallas import tpu as pltpu
from jax.experimental.pallas import tpu_sc as plsc

B = 16384
L = 200
D = 32
NC = 2
NS = 16
NW = NC * NS
BW = B // NW
C = 8
NCHUNK = BW // C


def _sc_embedding_bag(weights, indices_flat, table):
    mesh = plsc.VectorSubcoreMesh(
        core_axis_name="c", subcore_axis_name="s",
        num_cores=NC, num_subcores=NS,
    )

    @functools.partial(
        pl.kernel,
        out_type=jax.ShapeDtypeStruct((B, D), jnp.float32),
        mesh=mesh,
        scratch_types=[
            pltpu.VMEM((C * L,), jnp.int32),
            pltpu.VMEM((C, L), jnp.float32),
            pltpu.VMEM((C * L, D), jnp.float32),
            pltpu.VMEM((C, D), jnp.float32),
            pltpu.SemaphoreType.DMA,
        ],
        compiler_params=pltpu.CompilerParams(use_tc_tiling_on_sc=False),
    )
    def k(w_hbm, idx_hbm, tbl_hbm, out_hbm, idx_v, w_v, rows_v, out_v, sem):
        wid = lax.axis_index("s") * NC + lax.axis_index("c")
        base = wid * BW

        def chunk_body(g, carry):
            row0 = base + g * C
            pltpu.sync_copy(idx_hbm.at[pl.ds(row0 * L, C * L)], idx_v)
            pltpu.sync_copy(w_hbm.at[pl.ds(row0, C)], w_v)
            pltpu.async_copy(tbl_hbm.at[idx_v], rows_v, sem).wait()
            for c in range(C):
                def tok_body(g, acc):
                    a0, a1 = acc
                    wv = w_v[c, pl.ds(g * 16, 16)]
                    for j in range(16):
                        wgt = wv[j]
                        t = c * L + g * 16 + j
                        a0 = a0 + wgt * rows_v[t, pl.ds(0, 16)]
                        a1 = a1 + wgt * rows_v[t, pl.ds(16, 16)]
                    return (a0, a1)

                z = jnp.zeros((16,), jnp.float32)
                a0, a1 = lax.fori_loop(0, L // 16, tok_body, (z, z))
                wv = w_v[c, pl.ds(L - 16, 16)]
                for j in range(8, 16):
                    wgt = wv[j]
                    t = c * L + (L - 16) + j
                    a0 = a0 + wgt * rows_v[t, pl.ds(0, 16)]
                    a1 = a1 + wgt * rows_v[t, pl.ds(16, 16)]
                out_v[c, pl.ds(0, 16)] = jnp.maximum(a0, 0.0)
                out_v[c, pl.ds(16, 16)] = jnp.maximum(a1, 0.0)
            pltpu.sync_copy(out_v, out_hbm.at[pl.ds(row0, C)])
            return carry

        lax.fori_loop(0, NCHUNK, chunk_body, 0)

    return k(weights, indices_flat, table)


def kernel(weights, indices, table):
    idx = indices.astype(jnp.int32).reshape(B * L)
    return _sc_embedding_bag(weights, idx, table)


# speedup vs baseline: 15.9504x; 1.2619x over previous
import functools

import jax
import jax.numpy as jnp
from jax import lax
from jax.experimental import pallas as pl
from jax.experimental.pallas import tpu as pltpu
from jax.experimental.pallas import tpu_sc as plsc

B = 16384
L = 200
D = 32
NC = 2
NS = 16
NW = NC * NS
BW = B // NW
C = 8
NCHUNK = BW // C
NGRP = L // 16


def _sc_embedding_bag(weights, indices_flat, table):
    mesh = plsc.VectorSubcoreMesh(
        core_axis_name="c", subcore_axis_name="s",
        num_cores=NC, num_subcores=NS,
    )

    @functools.partial(
        pl.kernel,
        out_type=jax.ShapeDtypeStruct((B, D), jnp.float32),
        mesh=mesh,
        scratch_types=[
            pltpu.VMEM((2, C * L), jnp.int32),
            pltpu.VMEM((4, C, L), jnp.float32),
            pltpu.VMEM((2, C * L, D), jnp.float32),
            pltpu.VMEM((BW, D), jnp.float32),
            [pltpu.SemaphoreType.DMA] * 2,
            [pltpu.SemaphoreType.DMA] * 2,
            [pltpu.SemaphoreType.DMA] * 4,
        ],
        compiler_params=pltpu.CompilerParams(use_tc_tiling_on_sc=False),
    )
    def k(w_hbm, idx_hbm, tbl_hbm, out_hbm,
          idx_v, w_v, rows_v, out_v, sem_g, sem_i, sem_w):
        wid = lax.axis_index("s") * NC + lax.axis_index("c")
        base = wid * BW

        def issue_iw(g, pi, pw):
            gc = jnp.minimum(g, NCHUNK - 1)
            row0 = base + gc * C
            pltpu.async_copy(idx_hbm.at[pl.ds(row0 * L, C * L)],
                             idx_v.at[pi], sem_i[pi])
            pltpu.async_copy(w_hbm.at[pl.ds(row0, C)], w_v.at[pw], sem_w[pw])

        def wait_iw(pi, pw):
            pltpu.make_async_copy(idx_hbm.at[pl.ds(0, C * L)],
                                  idx_v.at[pi], sem_i[pi]).wait()
            pltpu.make_async_copy(w_hbm.at[pl.ds(0, C)],
                                  w_v.at[pw], sem_w[pw]).wait()

        def issue_gather(p):
            pltpu.async_copy(tbl_hbm.at[idx_v.at[p]], rows_v.at[p], sem_g[p])

        def wait_gather(p):
            pltpu.make_async_copy(tbl_hbm.at[idx_v.at[p]],
                                  rows_v.at[p], sem_g[p]).wait()

        def compute(g, p, pw):
            lrow0 = g * C
            for c in range(C):
                def tok_body(t16, acc):
                    a0, a1 = acc
                    wv = w_v[pw, c, pl.ds(t16 * 16, 16)]
                    for j in range(16):
                        wgt = wv[j]
                        t = c * L + t16 * 16 + j
                        a0 = a0 + wgt * rows_v[p, t, pl.ds(0, 16)]
                        a1 = a1 + wgt * rows_v[p, t, pl.ds(16, 16)]
                    return (a0, a1)

                z = jnp.zeros((16,), jnp.float32)
                a0, a1 = lax.fori_loop(0, NGRP, tok_body, (z, z))
                wv = w_v[pw, c, pl.ds(L - 16, 16)]
                for j in range(8, 16):
                    wgt = wv[j]
                    t = c * L + (L - 16) + j
                    a0 = a0 + wgt * rows_v[p, t, pl.ds(0, 16)]
                    a1 = a1 + wgt * rows_v[p, t, pl.ds(16, 16)]
                out_v[lrow0 + c, pl.ds(0, 16)] = jnp.maximum(a0, 0.0)
                out_v[lrow0 + c, pl.ds(16, 16)] = jnp.maximum(a1, 0.0)

        def step(g, kmod):
            p = kmod % 2
            wait_iw(1 - p, (kmod + 1) % 4)
            issue_gather(1 - p)
            wait_gather(p)
            issue_iw(g + 2, p, (kmod + 2) % 4)
            compute(g, p, kmod % 4)

        pltpu.sync_copy(idx_hbm.at[pl.ds(base * L, C * L)], idx_v.at[0])
        pltpu.sync_copy(w_hbm.at[pl.ds(base, C)], w_v.at[0])
        issue_gather(0)
        issue_iw(1, 1, 1)

        def quad_body(i, carry):
            for kk in range(4):
                step(4 * i + kk, kk)
            return carry

        lax.fori_loop(0, NCHUNK // 4, quad_body, 0)

        wait_gather(0)
        wait_iw(1, 1)
        pltpu.sync_copy(out_v, out_hbm.at[pl.ds(base, BW)])

    return k(weights, indices_flat, table)


def kernel(weights, indices, table):
    idx = indices.astype(jnp.int32).reshape(B * L)
    return _sc_embedding_bag(weights, idx, table)
